# Initial kernel scaffold; baseline (speedup 1.0000x reference)
import jax
import jax.numpy as jnp
from jax.experimental import pallas as pl


def kernel(x, edge_index, edge_attr, W_int, b_int, gamma, beta, W_res, b_res):
    raise NotImplementedError("write your pallas kernel here")


# speedup vs baseline: 3.1355x; 3.1355x over previous
import jax
import jax.numpy as jnp
from jax import lax
from jax.experimental import pallas as pl
from jax.experimental.pallas import tpu as pltpu
from jax.experimental.pallas import tpu_sc as plsc

N = 10000
E = 320000
D = 128
DE = 16
BN_EPS = 1e-5

NC = 2
NS = 16
NW = NC * NS
EPT = E // NW
C = 80
NCHUNK = EPT // C
NP = 10240
RPT = NP // NS
RSUB = 80


def _mm_bias_body(a_ref, w_ref, b_ref, o_ref):
    o_ref[...] = (
        jnp.dot(a_ref[...], w_ref[...], preferred_element_type=jnp.float32)
        + b_ref[...]
    )


def _edge_linear(edge_attr, w_et, b_int):
    blk = 8000
    grid = E // blk
    return pl.pallas_call(
        _mm_bias_body,
        grid=(grid,),
        in_specs=[
            pl.BlockSpec((blk, DE), lambda i: (i, 0)),
            pl.BlockSpec((DE, D), lambda i: (0, 0)),
            pl.BlockSpec((1, D), lambda i: (0, 0)),
        ],
        out_specs=pl.BlockSpec((blk, D), lambda i: (i, 0)),
        out_shape=jax.ShapeDtypeStruct((E, D), jnp.float32),
    )(edge_attr, w_et, b_int.reshape(1, D))


def _node_table(x, w_at):
    def body(x_ref, w_ref, o_ref):
        o_ref[...] = jnp.dot(
            x_ref[...], w_ref[...], preferred_element_type=jnp.float32
        )
    return pl.pallas_call(
        body,
        out_shape=jax.ShapeDtypeStruct((N, D), jnp.float32),
    )(x, w_at)


def _combine_body(p_ref, cnt_ref, x_ref, w_ref, br_ref, g_ref, bt_ref, o_ref):
    p = p_ref[...]
    s = p[0] + p[1]
    cnt = jnp.sum(cnt_ref[...], axis=0)[:, None]
    scale = g_ref[...] * (1.0 / jnp.sqrt(1.0 + BN_EPS))
    mean = jnp.where(
        cnt > 0.0,
        s / jnp.maximum(cnt, 1.0) * scale + bt_ref[...],
        0.0,
    )
    res = (
        jnp.dot(x_ref[...], w_ref[...], preferred_element_type=jnp.float32)
        + br_ref[...]
    )
    o_ref[...] = mean + res


def _combine(partials, counts, x, w_rt, b_res, gamma, beta):
    blk = 1024
    grid = NP // blk
    return pl.pallas_call(
        _combine_body,
        grid=(grid,),
        in_specs=[
            pl.BlockSpec((NC, blk, D), lambda i: (0, i, 0)),
            pl.BlockSpec((NW, blk), lambda i: (0, i)),
            pl.BlockSpec((blk, D), lambda i: (i, 0)),
            pl.BlockSpec((D, D), lambda i: (0, 0)),
            pl.BlockSpec((1, D), lambda i: (0, 0)),
            pl.BlockSpec((1, D), lambda i: (0, 0)),
            pl.BlockSpec((1, D), lambda i: (0, 0)),
        ],
        out_specs=pl.BlockSpec((blk, D), lambda i: (i, 0)),
        out_shape=jax.ShapeDtypeStruct((N, D), jnp.float32),
    )(partials, counts, x, w_rt, b_res.reshape(1, D), gamma.reshape(1, D),
      beta.reshape(1, D))


def _sc_body(y_hbm, c_hbm, row_hbm, col_hbm, out_hbm, outcnt_hbm,
             acc_sh, ybuf, cbuf, ubuf, rowbuf, colbuf, cntbuf, sem):
    cid = lax.axis_index("c")
    sid = lax.axis_index("s")
    wid = sid * NC + cid

    zero16 = jnp.zeros((16,), jnp.float32)

    @pl.loop(0, RSUB)
    def _zero_rows(r):
        for k in range(D // 16):
            ubuf[r, pl.ds(k * 16, 16)] = zero16

    base = sid * RPT
    for j in range(RPT // RSUB):
        pltpu.sync_copy(ubuf, acc_sh.at[pl.ds(base + j * RSUB, RSUB)])

    @pl.loop(0, NP // 16)
    def _zero_cnt(r):
        cntbuf[pl.ds(r * 16, 16)] = zero16

    plsc.subcore_barrier()

    ept_base = wid * EPT

    @pl.loop(0, NCHUNK)
    def _chunk(i):
        eb = ept_base + i * C
        pltpu.sync_copy(row_hbm.at[pl.ds(eb, C)], rowbuf)
        pltpu.sync_copy(col_hbm.at[pl.ds(eb, C)], colbuf)
        pltpu.sync_copy(c_hbm.at[pl.ds(eb, C)], cbuf)
        pltpu.async_copy(y_hbm.at[rowbuf], ybuf, sem).wait()

        @pl.loop(0, C)
        def _rows(r):
            for k in range(D // 16):
                a = ybuf[r, pl.ds(k * 16, 16)]
                b = cbuf[r, pl.ds(k * 16, 16)]
                ubuf[r, pl.ds(k * 16, 16)] = jnp.maximum(a + b, 0.0)

        for j in range(C // 16):
            cv = colbuf[pl.ds(j * 16, 16)]
            occ, last = plsc.scan_count(cv)
            plsc.addupdate_scatter(
                cntbuf, [cv], occ.astype(jnp.float32), mask=last
            )

        pltpu.sync_copy(ubuf, acc_sh.at[colbuf], add=True)

    pltpu.sync_copy(cntbuf, outcnt_hbm.at[wid])

    plsc.subcore_barrier()

    for j in range(RPT // RSUB):
        rb = base + j * RSUB
        pltpu.sync_copy(acc_sh.at[pl.ds(rb, RSUB)], ubuf)
        pltpu.sync_copy(ubuf, out_hbm.at[cid, pl.ds(rb, RSUB)])


def _sc_scatter(y, c, row, col):
    mesh = plsc.VectorSubcoreMesh(core_axis_name="c", subcore_axis_name="s")
    f = pl.kernel(
        _sc_body,
        out_type=(
            jax.ShapeDtypeStruct((NC, NP, D), jnp.float32),
            jax.ShapeDtypeStruct((NW, NP), jnp.float32),
        ),
        mesh=mesh,
        scratch_types=[
            pltpu.VMEM_SHARED((NP, D), jnp.float32),
            pltpu.VMEM((C, D), jnp.float32),
            pltpu.VMEM((C, D), jnp.float32),
            pltpu.VMEM((C, D), jnp.float32),
            pltpu.VMEM((C,), jnp.int32),
            pltpu.VMEM((C,), jnp.int32),
            pltpu.VMEM((NP,), jnp.float32),
            pltpu.SemaphoreType.DMA,
        ],
        compiler_params=pltpu.CompilerParams(needs_layout_passes=False),
    )
    return f(y, c, row, col)


def kernel(x, edge_index, edge_attr, W_int, b_int, gamma, beta, W_res, b_res):
    x = x.astype(jnp.float32)
    edge_attr = edge_attr.astype(jnp.float32)
    w_at = W_int[:, :D].T
    w_et = W_int[:, D:].T
    row = edge_index[0]
    col = edge_index[1]

    y = _node_table(x, w_at)
    c = _edge_linear(edge_attr, w_et, b_int)
    partials, counts = _sc_scatter(y, c, row, col)
    return _combine(partials, counts, x, W_res.T, b_res, gamma, beta)


# speedup vs baseline: 5.1680x; 1.6482x over previous
import jax
import jax.numpy as jnp
from jax import lax
from jax.experimental import pallas as pl
from jax.experimental.pallas import tpu as pltpu
from jax.experimental.pallas import tpu_sc as plsc

N = 10000
E = 320000
D = 128
DE = 16
BN_EPS = 1e-5

NC = 2
NS = 16
NW = NC * NS
EPT = E // NW
C = 40
NCHUNK = EPT // C
NBUF = 3
NP = 10240
RPT = NP // NS
RSUB = C


def _mm_bias_body(a_ref, w_ref, b_ref, o_ref):
    o_ref[...] = (
        jnp.dot(a_ref[...], w_ref[...], preferred_element_type=jnp.float32)
        + b_ref[...]
    )


def _edge_linear(edge_attr, w_et, b_int):
    blk = 8000
    grid = E // blk
    return pl.pallas_call(
        _mm_bias_body,
        grid=(grid,),
        in_specs=[
            pl.BlockSpec((blk, DE), lambda i: (i, 0)),
            pl.BlockSpec((DE, D), lambda i: (0, 0)),
            pl.BlockSpec((1, D), lambda i: (0, 0)),
        ],
        out_specs=pl.BlockSpec((blk, D), lambda i: (i, 0)),
        out_shape=jax.ShapeDtypeStruct((E, D), jnp.float32),
    )(edge_attr, w_et, b_int.reshape(1, D))


def _node_table(x, w_at):
    def body(x_ref, w_ref, o_ref):
        o_ref[...] = jnp.dot(
            x_ref[...], w_ref[...], preferred_element_type=jnp.float32
        )
    return pl.pallas_call(
        body,
        out_shape=jax.ShapeDtypeStruct((N, D), jnp.float32),
    )(x, w_at)


def _combine_body(p_ref, cnt_ref, x_ref, w_ref, br_ref, g_ref, bt_ref, o_ref):
    p = p_ref[...]
    s = p[0] + p[1]
    cnt = jnp.sum(cnt_ref[...], axis=0)[:, None]
    scale = g_ref[...] * (1.0 / jnp.sqrt(1.0 + BN_EPS))
    mean = jnp.where(
        cnt > 0.0,
        s / jnp.maximum(cnt, 1.0) * scale + bt_ref[...],
        0.0,
    )
    res = (
        jnp.dot(x_ref[...], w_ref[...], preferred_element_type=jnp.float32)
        + br_ref[...]
    )
    o_ref[...] = mean + res


def _combine(partials, counts, x, w_rt, b_res, gamma, beta):
    blk = 1024
    grid = NP // blk
    return pl.pallas_call(
        _combine_body,
        grid=(grid,),
        in_specs=[
            pl.BlockSpec((NC, blk, D), lambda i: (0, i, 0)),
            pl.BlockSpec((NW, blk), lambda i: (0, i)),
            pl.BlockSpec((blk, D), lambda i: (i, 0)),
            pl.BlockSpec((D, D), lambda i: (0, 0)),
            pl.BlockSpec((1, D), lambda i: (0, 0)),
            pl.BlockSpec((1, D), lambda i: (0, 0)),
            pl.BlockSpec((1, D), lambda i: (0, 0)),
        ],
        out_specs=pl.BlockSpec((blk, D), lambda i: (i, 0)),
        out_shape=jax.ShapeDtypeStruct((N, D), jnp.float32),
    )(partials, counts, x, w_rt, b_res.reshape(1, D), gamma.reshape(1, D),
      beta.reshape(1, D))


def _sc_body(y_hbm, c_hbm, row_hbm, col_hbm, out_hbm, outcnt_hbm,
             acc_sh,
             ybuf0, ybuf1, ybuf2, cbuf0, cbuf1, cbuf2,
             rowbuf0, rowbuf1, rowbuf2, colbuf0, colbuf1, colbuf2,
             cntbuf,
             sl0, sl1, sl2, sg0, sg1, sg2, ss0, ss1, ss2):
    ybuf = (ybuf0, ybuf1, ybuf2)
    cbuf = (cbuf0, cbuf1, cbuf2)
    rowbuf = (rowbuf0, rowbuf1, rowbuf2)
    colbuf = (colbuf0, colbuf1, colbuf2)
    sem_l = (sl0, sl1, sl2)
    sem_g = (sg0, sg1, sg2)
    sem_s = (ss0, ss1, ss2)

    cid = lax.axis_index("c")
    sid = lax.axis_index("s")
    wid = sid * NC + cid

    zero16 = jnp.zeros((16,), jnp.float32)

    @pl.loop(0, RSUB)
    def _zero_rows(r):
        for k in range(D // 16):
            ybuf0[r, pl.ds(k * 16, 16)] = zero16

    base = sid * RPT
    for j in range(RPT // RSUB):
        pltpu.sync_copy(ybuf0, acc_sh.at[pl.ds(base + j * RSUB, RSUB)])

    @pl.loop(0, NP // 16)
    def _zero_cnt(r):
        cntbuf[pl.ds(r * 16, 16)] = zero16

    plsc.subcore_barrier()

    ept_base = wid * EPT
    tail_valid = lax.iota(jnp.int32, 16) >= (16 - (C % 16))

    def issue_loads(g, b):
        eb = ept_base + g * C
        pltpu.async_copy(row_hbm.at[pl.ds(eb, C)], rowbuf[b], sem_l[b])
        pltpu.async_copy(col_hbm.at[pl.ds(eb, C)], colbuf[b], sem_l[b])
        pltpu.async_copy(c_hbm.at[pl.ds(eb, C)], cbuf[b], sem_l[b])

    def wait_loads(g, b):
        eb = ept_base + g * C
        pltpu.make_async_copy(row_hbm.at[pl.ds(eb, C)], rowbuf[b], sem_l[b]).wait()
        pltpu.make_async_copy(col_hbm.at[pl.ds(eb, C)], colbuf[b], sem_l[b]).wait()
        pltpu.make_async_copy(c_hbm.at[pl.ds(eb, C)], cbuf[b], sem_l[b]).wait()

    def compute_scatter(b):
        @pl.loop(0, C)
        def _rows(r):
            for k in range(D // 16):
                a = ybuf[b][r, pl.ds(k * 16, 16)]
                v = cbuf[b][r, pl.ds(k * 16, 16)]
                ybuf[b][r, pl.ds(k * 16, 16)] = jnp.maximum(a + v, 0.0)

        for j in range(C // 16):
            cv = colbuf[b][pl.ds(j * 16, 16)]
            occ, last = plsc.scan_count(cv)
            plsc.addupdate_scatter(
                cntbuf, [cv], occ.astype(jnp.float32), mask=last
            )
        if C % 16:
            cv = colbuf[b][pl.ds(C - 16, 16)]
            occ, last = plsc.scan_count(cv, mask=tail_valid)
            plsc.addupdate_scatter(
                cntbuf, [cv], occ.astype(jnp.float32), mask=last
            )

        pltpu.async_copy(ybuf[b], acc_sh.at[colbuf[b]], sem_s[b], add=True)

    assert (NCHUNK + 2) % NBUF == 0

    @pl.loop(0, (NCHUNK + 2) // NBUF)
    def _steady(t):
        for bb in range(NBUF):
            s = t * NBUF + bb

            @pl.when(s >= 3)
            def _():
                b = bb
                pltpu.make_async_copy(
                    ybuf[b], acc_sh.at[colbuf[b]], sem_s[b]
                ).wait()

            @pl.when(s < NCHUNK)
            def _():
                issue_loads(s, bb)

            @pl.when(jnp.logical_and(s >= 1, s <= NCHUNK))
            def _():
                b = (bb - 1) % NBUF
                wait_loads(s - 1, b)
                pltpu.async_copy(y_hbm.at[rowbuf[b]], ybuf[b], sem_g[b])

            @pl.when(s >= 2)
            def _():
                b = (bb - 2) % NBUF
                pltpu.make_async_copy(
                    y_hbm.at[rowbuf[b]], ybuf[b], sem_g[b]
                ).wait()
                compute_scatter(b)

    blast = (NCHUNK - 1) % NBUF
    pltpu.make_async_copy(
        ybuf[blast], acc_sh.at[colbuf[blast]], sem_s[blast]
    ).wait()

    pltpu.sync_copy(cntbuf, outcnt_hbm.at[wid])

    plsc.subcore_barrier()

    for j in range(RPT // RSUB):
        rb = base + j * RSUB
        pltpu.sync_copy(acc_sh.at[pl.ds(rb, RSUB)], ybuf0)
        pltpu.sync_copy(ybuf0, out_hbm.at[cid, pl.ds(rb, RSUB)])


def _sc_scatter(y, c, row, col):
    mesh = plsc.VectorSubcoreMesh(core_axis_name="c", subcore_axis_name="s")
    f = pl.kernel(
        _sc_body,
        out_type=(
            jax.ShapeDtypeStruct((NC, NP, D), jnp.float32),
            jax.ShapeDtypeStruct((NW, NP), jnp.float32),
        ),
        mesh=mesh,
        scratch_types=[
            pltpu.VMEM_SHARED((NP, D), jnp.float32),
            pltpu.VMEM((C, D), jnp.float32),
            pltpu.VMEM((C, D), jnp.float32),
            pltpu.VMEM((C, D), jnp.float32),
            pltpu.VMEM((C, D), jnp.float32),
            pltpu.VMEM((C, D), jnp.float32),
            pltpu.VMEM((C, D), jnp.float32),
            pltpu.VMEM((C,), jnp.int32),
            pltpu.VMEM((C,), jnp.int32),
            pltpu.VMEM((C,), jnp.int32),
            pltpu.VMEM((C,), jnp.int32),
            pltpu.VMEM((C,), jnp.int32),
            pltpu.VMEM((C,), jnp.int32),
            pltpu.VMEM((NP,), jnp.float32),
            pltpu.SemaphoreType.DMA,
            pltpu.SemaphoreType.DMA,
            pltpu.SemaphoreType.DMA,
            pltpu.SemaphoreType.DMA,
            pltpu.SemaphoreType.DMA,
            pltpu.SemaphoreType.DMA,
            pltpu.SemaphoreType.DMA,
            pltpu.SemaphoreType.DMA,
            pltpu.SemaphoreType.DMA,
        ],
        compiler_params=pltpu.CompilerParams(needs_layout_passes=False),
    )
    return f(y, c, row, col)


def kernel(x, edge_index, edge_attr, W_int, b_int, gamma, beta, W_res, b_res):
    x = x.astype(jnp.float32)
    edge_attr = edge_attr.astype(jnp.float32)
    w_at = W_int[:, :D].T
    w_et = W_int[:, D:].T
    row = edge_index[0]
    col = edge_index[1]

    y = _node_table(x, w_at)
    c = _edge_linear(edge_attr, w_et, b_int)
    partials, counts = _sc_scatter(y, c, row, col)
    return _combine(partials, counts, x, W_res.T, b_res, gamma, beta)
